# speedup vs baseline: 1.1517x; 1.1517x over previous
import functools

import jax
import jax.numpy as jnp
from jax import lax
from jax.experimental import pallas as pl
from jax.experimental.pallas import tpu as pltpu
from jax.experimental.pallas import tpu_sc as plsc

_N = 100000
_D = 256
_NC, _NS = 2, 16
_NW = _NC * _NS
_CH = 128
_NCH = 25
_PER_W = _CH * _NCH
_B_PAD = _NW * _PER_W
_FULL = (_N // _CH) * _CH
_TAIL = _N - _FULL

_mesh = plsc.VectorSubcoreMesh(core_axis_name="c", subcore_axis_name="s")


@functools.partial(
    pl.kernel,
    mesh=_mesh,
    out_type=jax.ShapeDtypeStruct((_N, _D), jnp.float32),
    scratch_types=[
        pltpu.VMEM((_NCH, _CH), jnp.int32),
        pltpu.VMEM((_CH, _D), jnp.float32),
        pltpu.SemaphoreType.DMA,
    ],
)
def _emb_gather(idx_hbm, emb_hbm, out_hbm, idx_v, rows_v, sem):
    wid = lax.axis_index("s") * _NC + lax.axis_index("c")
    pltpu.sync_copy(idx_hbm.at[wid], idx_v)
    base = wid * _PER_W

    def body(j, carry):
        pltpu.async_copy(emb_hbm.at[idx_v.at[j]], rows_v, sem).wait()
        rbase = base + j * _CH

        @pl.when(rbase + _CH <= _N)
        def _full_write():
            pltpu.sync_copy(rows_v, out_hbm.at[pl.ds(rbase, _CH)])

        @pl.when(rbase == _FULL)
        def _tail_write():
            pltpu.sync_copy(rows_v.at[pl.ds(0, _TAIL)],
                            out_hbm.at[pl.ds(_FULL, _TAIL)])

        return carry

    lax.fori_loop(0, _NCH, body, 0)


def kernel(node_val, emb):
    idx = node_val.reshape(-1).astype(jnp.int32)
    idx = jnp.pad(idx, (0, _B_PAD - _N))
    return _emb_gather(idx.reshape(_NW, _NCH, _CH), emb)


# speedup vs baseline: 1.1979x; 1.0401x over previous
import functools

import jax
import jax.numpy as jnp
from jax import lax
from jax.experimental import pallas as pl
from jax.experimental.pallas import tpu as pltpu
from jax.experimental.pallas import tpu_sc as plsc

_N = 100000
_D = 256
_NC, _NS = 2, 16
_NW = _NC * _NS
_CH = 128
_NCH = 25
_PER_W = _CH * _NCH
_B_PAD = _NW * _PER_W
_FULL = (_N // _CH) * _CH
_TAIL = _N - _FULL

_mesh = plsc.VectorSubcoreMesh(core_axis_name="c", subcore_axis_name="s")


@functools.partial(
    pl.kernel,
    mesh=_mesh,
    out_type=jax.ShapeDtypeStruct((_N, _D), jnp.float32),
    scratch_types=[
        pltpu.VMEM((_NCH, _CH), jnp.int32),
        pltpu.VMEM((2, _CH, _D), jnp.float32),
        pltpu.SemaphoreType.DMA,
        pltpu.SemaphoreType.DMA,
    ],
)
def _emb_gather(idx_hbm, emb_hbm, out_hbm, idx_v, rows_v, sem0, sem1):
    wid = lax.axis_index("s") * _NC + lax.axis_index("c")
    pltpu.sync_copy(idx_hbm.at[wid], idx_v)
    base = wid * _PER_W
    sems = (sem0, sem1)

    hg = [None, None]
    hg[0] = pltpu.async_copy(emb_hbm.at[idx_v.at[0]], rows_v.at[0], sems[0])
    for j in range(_NCH):
        b = j & 1
        if j + 1 < _NCH:
            hg[1 - b] = pltpu.async_copy(emb_hbm.at[idx_v.at[j + 1]],
                                         rows_v.at[1 - b], sems[1 - b])
        hg[b].wait()
        rbase = base + j * _CH

        @pl.when(rbase + _CH <= _N)
        def _full_write(b=b, rbase=rbase):
            pltpu.sync_copy(rows_v.at[b], out_hbm.at[pl.ds(rbase, _CH)])

        @pl.when(rbase == _FULL)
        def _tail_write(b=b):
            pltpu.sync_copy(rows_v.at[b].at[pl.ds(0, _TAIL)],
                            out_hbm.at[pl.ds(_FULL, _TAIL)])


def kernel(node_val, emb):
    idx = node_val.reshape(-1).astype(jnp.int32)
    idx = jnp.pad(idx, (0, _B_PAD - _N))
    return _emb_gather(idx.reshape(_NW, _NCH, _CH), emb)


# speedup vs baseline: 1.9530x; 1.6303x over previous
import functools

import jax
import jax.numpy as jnp
import numpy as np
from jax import lax
from jax.experimental import pallas as pl
from jax.experimental.pallas import tpu as pltpu
from jax.experimental.pallas import tpu_sc as plsc

_N = 100000
_D = 256
_NC, _NS = 2, 16
_NW = _NC * _NS
_CH = 128
_FAST_NCH = 35
_SLOW_NCH = 14
_FAST_CORE = 0
_TOT_CH = _NS * (_FAST_NCH + _SLOW_NCH)
_FULL = (_N // _CH) * _CH
_TAIL = _N - _FULL


def _chunk_of(w, j):
    c, s = divmod(w, _NS)
    if c == _FAST_CORE:
        first, nch = s * _FAST_NCH, _FAST_NCH
    else:
        first, nch = _NS * _FAST_NCH + s * _SLOW_NCH, _SLOW_NCH
    return first + j if j < nch else _TOT_CH


_ROWS_MAP = np.array([_chunk_of(w, j)
                      for w in range(_NW) for j in range(_FAST_NCH)],
                     dtype=np.int32)

_mesh = plsc.VectorSubcoreMesh(core_axis_name="c", subcore_axis_name="s")


@functools.partial(
    pl.kernel,
    mesh=_mesh,
    out_type=jax.ShapeDtypeStruct((_N, _D), jnp.float32),
    scratch_types=[
        pltpu.VMEM((_FAST_NCH, _CH), jnp.int32),
        pltpu.VMEM((2, _CH, _D), jnp.float32),
        pltpu.SemaphoreType.DMA,
        pltpu.SemaphoreType.DMA,
    ],
)
def _emb_gather(idx_hbm, emb_hbm, out_hbm, idx_v, rows_v, sem0, sem1):
    cid = lax.axis_index("c")
    sid = lax.axis_index("s")
    wid = cid * _NS + sid
    on_fast = cid == _FAST_CORE
    my_nch = jnp.where(on_fast, _FAST_NCH, _SLOW_NCH)
    chunk0 = jnp.where(on_fast, sid * _FAST_NCH,
                       _NS * _FAST_NCH + sid * _SLOW_NCH)
    pltpu.sync_copy(idx_hbm.at[wid], idx_v)
    sems = (sem0, sem1)

    pltpu.async_copy(emb_hbm.at[idx_v.at[0]], rows_v.at[0], sems[0])
    for j in range(_FAST_NCH):
        b = j & 1

        if j + 1 < _FAST_NCH:
            @pl.when(j + 1 < my_nch)
            def _issue_next(b=b, j=j):
                pltpu.async_copy(emb_hbm.at[idx_v.at[j + 1]],
                                 rows_v.at[1 - b], sems[1 - b])

        @pl.when(j < my_nch)
        def _wait_and_write(b=b, j=j):
            pltpu.make_async_copy(emb_hbm.at[idx_v.at[j]],
                                  rows_v.at[b], sems[b]).wait()
            rbase = (chunk0 + j) * _CH

            @pl.when(rbase + _CH <= _N)
            def _full_write():
                pltpu.sync_copy(rows_v.at[b], out_hbm.at[pl.ds(rbase, _CH)])

            @pl.when(rbase == _FULL)
            def _tail_write():
                pltpu.sync_copy(rows_v.at[b].at[pl.ds(0, _TAIL)],
                                out_hbm.at[pl.ds(_FULL, _TAIL)])


def kernel(node_val, emb):
    idx = node_val.reshape(-1).astype(jnp.int32)
    idx = jnp.pad(idx, (0, (_TOT_CH + 1) * _CH - _N))
    idx = jnp.take(idx.reshape(_TOT_CH + 1, _CH), _ROWS_MAP, axis=0)
    return _emb_gather(idx.reshape(_NW, _FAST_NCH, _CH), emb)


# speedup vs baseline: 2.2613x; 1.1579x over previous
import functools

import jax
import jax.numpy as jnp
from jax import lax
from jax.experimental import pallas as pl
from jax.experimental.pallas import tpu as pltpu
from jax.experimental.pallas import tpu_sc as plsc

_N = 100000
_D = 256
_NC, _NS = 2, 16
_CH = 128
_FAST_NCH = 34
_SLOW_NCH = 15
_FAST_CORE = 0
_TOT_CH = _NS * (_FAST_NCH + _SLOW_NCH)
_IDX_PAD = (_TOT_CH + 1) * _CH
_FULL = (_N // _CH) * _CH
_TAIL = _N - _FULL

_mesh = plsc.VectorSubcoreMesh(core_axis_name="c", subcore_axis_name="s")


@functools.partial(
    pl.kernel,
    mesh=_mesh,
    out_type=jax.ShapeDtypeStruct((_N, _D), jnp.float32),
    scratch_types=[
        pltpu.VMEM((_FAST_NCH * _CH,), jnp.int32),
        pltpu.VMEM((2, _CH, _D), jnp.float32),
        pltpu.SemaphoreType.DMA,
        pltpu.SemaphoreType.DMA,
    ],
)
def _emb_gather(idx_hbm, emb_hbm, out_hbm, idx_v, rows_v, sem0, sem1):
    cid = lax.axis_index("c")
    sid = lax.axis_index("s")
    on_fast = cid == _FAST_CORE
    my_nch = jnp.where(on_fast, _FAST_NCH, _SLOW_NCH)
    chunk0 = jnp.where(on_fast, sid * _FAST_NCH,
                       _NS * _FAST_NCH + sid * _SLOW_NCH)

    @pl.when(on_fast)
    def _stage_fast():
        pltpu.sync_copy(idx_hbm.at[pl.ds(chunk0 * _CH, _FAST_NCH * _CH)],
                        idx_v)

    @pl.when(jnp.logical_not(on_fast))
    def _stage_slow():
        pltpu.sync_copy(idx_hbm.at[pl.ds(chunk0 * _CH, _SLOW_NCH * _CH)],
                        idx_v.at[pl.ds(0, _SLOW_NCH * _CH)])

    sems = (sem0, sem1)

    pltpu.async_copy(emb_hbm.at[idx_v.at[pl.ds(0, _CH)]], rows_v.at[0],
                     sems[0])
    for j in range(_FAST_NCH):
        b = j & 1

        if j + 1 < _FAST_NCH:
            @pl.when(j + 1 < my_nch)
            def _issue_next(b=b, j=j):
                pltpu.async_copy(
                    emb_hbm.at[idx_v.at[pl.ds((j + 1) * _CH, _CH)]],
                    rows_v.at[1 - b], sems[1 - b])

        @pl.when(j < my_nch)
        def _wait_and_write(b=b, j=j):
            pltpu.make_async_copy(emb_hbm.at[idx_v.at[pl.ds(j * _CH, _CH)]],
                                  rows_v.at[b], sems[b]).wait()
            rbase = (chunk0 + j) * _CH

            @pl.when(rbase + _CH <= _N)
            def _full_write():
                pltpu.sync_copy(rows_v.at[b], out_hbm.at[pl.ds(rbase, _CH)])

            @pl.when(rbase == _FULL)
            def _tail_write():
                pltpu.sync_copy(rows_v.at[b].at[pl.ds(0, _TAIL)],
                                out_hbm.at[pl.ds(_FULL, _TAIL)])


def kernel(node_val, emb):
    idx = node_val.reshape(-1).astype(jnp.int32)
    idx = jnp.pad(idx, (0, _IDX_PAD - _N))
    return _emb_gather(idx, emb)


# speedup vs baseline: 2.3996x; 1.0612x over previous
import functools

import jax
import jax.numpy as jnp
from jax import lax
from jax.experimental import pallas as pl
from jax.experimental.pallas import tpu as pltpu
from jax.experimental.pallas import tpu_sc as plsc

_N = 100000
_D = 256
_NC, _NS = 2, 16
_CH = 128
_FAST_NCH = 34
_SLOW_NCH = 15
_FAST_CORE = 0
_TOT_CH = _NS * (_FAST_NCH + _SLOW_NCH)
_IDX_PAD = (_TOT_CH + 1) * _CH
_FULL = (_N // _CH) * _CH
_TAIL = _N - _FULL

_mesh = plsc.VectorSubcoreMesh(core_axis_name="c", subcore_axis_name="s")


@functools.partial(
    pl.kernel,
    mesh=_mesh,
    out_type=jax.ShapeDtypeStruct((_N, _D), jnp.float32),
    scratch_types=[
        pltpu.VMEM((_FAST_NCH * _CH,), jnp.int32),
        pltpu.VMEM((3, _CH, _D), jnp.float32),
        pltpu.SemaphoreType.DMA,
        pltpu.SemaphoreType.DMA,
        pltpu.SemaphoreType.DMA,
    ],
)
def _emb_gather(idx_hbm, emb_hbm, out_hbm, idx_v, rows_v, sem0, sem1, sem2):
    cid = lax.axis_index("c")
    sid = lax.axis_index("s")
    on_fast = cid == _FAST_CORE
    my_nch = jnp.where(on_fast, _FAST_NCH, _SLOW_NCH)
    chunk0 = jnp.where(on_fast, sid * _FAST_NCH,
                       _NS * _FAST_NCH + sid * _SLOW_NCH)

    @pl.when(on_fast)
    def _stage_fast():
        pltpu.sync_copy(idx_hbm.at[pl.ds(chunk0 * _CH, _FAST_NCH * _CH)],
                        idx_v)

    @pl.when(jnp.logical_not(on_fast))
    def _stage_slow():
        pltpu.sync_copy(idx_hbm.at[pl.ds(chunk0 * _CH, _SLOW_NCH * _CH)],
                        idx_v.at[pl.ds(0, _SLOW_NCH * _CH)])

    sems = (sem0, sem1, sem2)

    pltpu.async_copy(emb_hbm.at[idx_v.at[pl.ds(0, _CH)]], rows_v.at[0],
                     sems[0])

    @pl.when(1 < my_nch)
    def _issue_second():
        pltpu.async_copy(emb_hbm.at[idx_v.at[pl.ds(_CH, _CH)]],
                         rows_v.at[1], sems[1])

    for j in range(_FAST_NCH):
        b = j % 3

        if j + 2 < _FAST_NCH:
            @pl.when(j + 2 < my_nch)
            def _issue_next(j=j):
                b2 = (j + 2) % 3
                pltpu.async_copy(
                    emb_hbm.at[idx_v.at[pl.ds((j + 2) * _CH, _CH)]],
                    rows_v.at[b2], sems[b2])

        @pl.when(j < my_nch)
        def _wait_and_write(b=b, j=j):
            pltpu.make_async_copy(emb_hbm.at[idx_v.at[pl.ds(j * _CH, _CH)]],
                                  rows_v.at[b], sems[b]).wait()
            rbase = (chunk0 + j) * _CH

            @pl.when(rbase + _CH <= _N)
            def _full_write():
                pltpu.sync_copy(rows_v.at[b], out_hbm.at[pl.ds(rbase, _CH)])

            @pl.when(rbase == _FULL)
            def _tail_write():
                pltpu.sync_copy(rows_v.at[b].at[pl.ds(0, _TAIL)],
                                out_hbm.at[pl.ds(_FULL, _TAIL)])


def kernel(node_val, emb):
    idx = node_val.reshape(-1).astype(jnp.int32)
    idx = jnp.pad(idx, (0, _IDX_PAD - _N))
    return _emb_gather(idx, emb)
